# baseline (speedup 1.0000x reference)
import functools

import jax
import jax.numpy as jnp
from jax import lax
from jax.experimental import pallas as pl
from jax.experimental.pallas import tpu as pltpu
from jax.experimental.pallas import tpu_sc as plsc

CTX = 50
EMB = 64
VOCAB = 100000
HID = 128
BN = 3200
NBLK = 32
NP = 4
NSTEP = NBLK // NP
VPAD = NBLK * BN
LASTB = pl.cdiv(VOCAB, BN) - 1
NEG = -1e30

_NT = (((1,), (1,)), ((), ()))


_sc_mesh = plsc.VectorSubcoreMesh(core_axis_name="c", subcore_axis_name="s")


@functools.partial(
    pl.kernel,
    mesh=_sc_mesh,
    out_type=jax.ShapeDtypeStruct((CTX, 2 * EMB), jnp.float32),
    scratch_types=[
        pltpu.VMEM((CTX,), jnp.int32),
        pltpu.VMEM((CTX, 2 * EMB), jnp.float32),
        pltpu.SemaphoreType.DMA,
    ],
)
def _sc_gather(idx_hbm, table_hbm, out_hbm, idx_v, rows_v, sem):
    wid = lax.axis_index("s") * 2 + lax.axis_index("c")

    @pl.when(wid == 0)
    def _():
        pltpu.sync_copy(idx_hbm, idx_v)
        pltpu.async_copy(table_hbm.at[idx_v], rows_v, sem).wait()
        pltpu.sync_copy(rows_v, out_hbm)


def _fwd_body(idx_ref, rows_ref, w1_ref, b1_ref, *rest):
    w2_refs = rest[:NP]
    b2_ref, out_ref, hid_s = rest[NP:]
    i = pl.program_id(0)

    @pl.when(i == 0)
    def _hidden():
        pairs = rows_ref[...]
        r_io = jax.lax.broadcasted_iota(jnp.int32, (2 * EMB, EMB), 0)
        c_io = jax.lax.broadcasted_iota(jnp.int32, (2 * EMB, EMB), 1)
        acc = jnp.zeros((1, HID), jnp.float32)
        for j in range(CTX):
            par = idx_ref[j] % 2
            sel = (r_io == c_io + EMB * par).astype(jnp.float32)
            emb_j = jnp.dot(pairs[j:j + 1, :], sel,
                            preferred_element_type=jnp.float32)
            acc = acc + jnp.dot(emb_j, w1_ref[j],
                                preferred_element_type=jnp.float32)
        hid_s[...] = jnp.maximum(acc + b1_ref[...], 0.0)

    hid = hid_s[...]
    for p, w2_ref in enumerate(w2_refs):
        b = i * NP + p
        logits = jax.lax.dot_general(
            hid, w2_ref[...], _NT,
            preferred_element_type=jnp.float32) + b2_ref[b]
        col = b * BN + jax.lax.broadcasted_iota(jnp.int32, (1, BN), 1)
        logits = jnp.where(col < VOCAB, logits, NEG)
        out_ref[b, :] = logits[0]

    @pl.when(i == NSTEP - 1)
    def _log_softmax():
        x = out_ref[...]
        m = jnp.max(x)
        lse = m + jnp.log(jnp.sum(jnp.exp(x - m)))
        out_ref[...] = x - lse


@jax.jit
def _fwd(idx, rows, w1, b1, w2t, b2pad):
    grid_spec = pltpu.PrefetchScalarGridSpec(
        num_scalar_prefetch=1,
        grid=(NSTEP,),
        in_specs=[
            pl.BlockSpec((CTX, 2 * EMB), lambda i, ref: (0, 0)),
            pl.BlockSpec((CTX, EMB, HID), lambda i, ref: (0, 0, 0)),
            pl.BlockSpec((1, HID), lambda i, ref: (0, 0)),
            *[pl.BlockSpec(
                  (BN, HID),
                  functools.partial(
                      lambda p, i, ref: (jnp.minimum(NP * i + p, LASTB), 0),
                      p))
              for p in range(NP)],
            pl.BlockSpec((NBLK, BN), lambda i, ref: (0, 0)),
        ],
        out_specs=pl.BlockSpec((NBLK, BN), lambda i, ref: (0, 0)),
        scratch_shapes=[
            pltpu.VMEM((1, HID), jnp.float32),
        ],
    )
    return pl.pallas_call(
        _fwd_body,
        grid_spec=grid_spec,
        out_shape=jax.ShapeDtypeStruct((NBLK, BN), jnp.float32),
        compiler_params=pltpu.CompilerParams(
            dimension_semantics=("arbitrary",),
        ),
    )(idx, rows, w1, b1, *([w2t] * NP), b2pad)


def kernel(context_words, emb_table, W1, b1, W2, b2):
    idx = context_words.astype(jnp.int32)
    rows = _sc_gather(idx // 2, emb_table.reshape(VOCAB // 2, 2 * EMB))
    w2t = W2.T
    w1 = W1.reshape(CTX, EMB, HID)
    b1r = b1.reshape(1, HID)
    b2pad = jnp.pad(b2, (0, VPAD - VOCAB)).reshape(NBLK, BN)
    out = _fwd(idx, rows, w1, b1r, w2t, b2pad)
    return out.reshape(1, VPAD)[:, :VOCAB]


# speedup vs baseline: 3.3285x; 3.3285x over previous
import functools

import jax
import jax.numpy as jnp
from jax.experimental import pallas as pl
from jax.experimental.pallas import tpu as pltpu

CTX = 50
EMB = 64
VOCAB = 100000
HID = 128
BN = 2560
NBLK = 40
NP = 4
NSTEP = NBLK // NP
VPAD = NBLK * BN
LASTB = pl.cdiv(VOCAB, BN) - 1
NEG = -1e30

_NT = (((1,), (1,)), ((), ()))


def _fwd_body(idx_ref, embt_hbm, w1_ref, b1_ref, *rest):
    w2_refs = rest[:NP]
    b2_ref, out_ref, embt_s, hid_s, sem = rest[NP:]
    i = pl.program_id(0)

    @pl.when(i == 0)
    def _gather_and_hidden():
        for j in range(CTX):
            off = pl.multiple_of(idx_ref[j] // 128 * 128, 128)
            pltpu.make_async_copy(embt_hbm.at[:, pl.ds(off, 128)],
                                  embt_s.at[j], sem).start()
        for j in range(CTX):
            off = pl.multiple_of(idx_ref[j] // 128 * 128, 128)
            pltpu.make_async_copy(embt_hbm.at[:, pl.ds(off, 128)],
                                  embt_s.at[j], sem).wait()
        lanes = jax.lax.broadcasted_iota(jnp.int32, (1, 128), 1)
        acc = jnp.zeros((1, HID), jnp.float32)
        for j in range(CTX):
            onehot = (lanes == idx_ref[j] % 128).astype(jnp.float32)
            emb_j = jax.lax.dot_general(
                onehot, embt_s[j], _NT,
                preferred_element_type=jnp.float32)
            acc = acc + jnp.dot(emb_j, w1_ref[j],
                                preferred_element_type=jnp.float32)
        hid_s[...] = jnp.maximum(acc + b1_ref[...], 0.0)

    hid = hid_s[...]
    for p, w2_ref in enumerate(w2_refs):
        b = i * NP + p
        logits = jax.lax.dot_general(
            hid, w2_ref[...], _NT,
            preferred_element_type=jnp.float32) + b2_ref[b]
        col = b * BN + jax.lax.broadcasted_iota(jnp.int32, (1, BN), 1)
        logits = jnp.where(col < VOCAB, logits, NEG)
        out_ref[b, :] = logits[0]

    @pl.when(i == NSTEP - 1)
    def _log_softmax():
        x = out_ref[...]
        m = jnp.max(x)
        lse = m + jnp.log(jnp.sum(jnp.exp(x - m)))
        out_ref[...] = x - lse


@jax.jit
def _fwd(idx, embt, w1, b1, w2t, b2pad):
    grid_spec = pltpu.PrefetchScalarGridSpec(
        num_scalar_prefetch=1,
        grid=(NSTEP,),
        in_specs=[
            pl.BlockSpec(memory_space=pl.ANY),
            pl.BlockSpec((CTX, EMB, HID), lambda i, ref: (0, 0, 0)),
            pl.BlockSpec((1, HID), lambda i, ref: (0, 0)),
            *[pl.BlockSpec(
                  (BN, HID),
                  functools.partial(
                      lambda p, i, ref: (jnp.minimum(NP * i + p, LASTB), 0),
                      p))
              for p in range(NP)],
            pl.BlockSpec((NBLK, BN), lambda i, ref: (0, 0)),
        ],
        out_specs=pl.BlockSpec((NBLK, BN), lambda i, ref: (0, 0)),
        scratch_shapes=[
            pltpu.VMEM((CTX, EMB, 128), jnp.float32),
            pltpu.VMEM((1, HID), jnp.float32),
            pltpu.SemaphoreType.DMA,
        ],
    )
    return pl.pallas_call(
        _fwd_body,
        grid_spec=grid_spec,
        out_shape=jax.ShapeDtypeStruct((NBLK, BN), jnp.float32),
        compiler_params=pltpu.CompilerParams(
            dimension_semantics=("arbitrary",),
        ),
    )(idx, embt, w1, b1, *([w2t] * NP), b2pad)


def kernel(context_words, emb_table, W1, b1, W2, b2):
    idx = context_words.astype(jnp.int32)
    embt = emb_table.T
    w2t = W2.T
    w1 = W1.reshape(CTX, EMB, HID)
    b1r = b1.reshape(1, HID)
    b2pad = jnp.pad(b2, (0, VPAD - VOCAB)).reshape(NBLK, BN)
    out = _fwd(idx, embt, w1, b1r, w2t, b2pad)
    return out.reshape(1, VPAD)[:, :VOCAB]


# speedup vs baseline: 3.3832x; 1.0164x over previous
import functools

import jax
import jax.numpy as jnp
from jax.experimental import pallas as pl
from jax.experimental.pallas import tpu as pltpu

CTX = 50
EMB = 64
VOCAB = 100000
HID = 128
BN = 3200
NBLK = 32
NP = 4
NSTEP = NBLK // NP
VPAD = NBLK * BN
LASTB = pl.cdiv(VOCAB, BN) - 1
NEG = -1e30

_NT = (((1,), (1,)), ((), ()))


def _fwd_body(idx_ref, embt_hbm, w1_ref, b1_ref, *rest):
    w2_refs = rest[:NP]
    b2_ref, out_ref, embt_s, hid_s, sem = rest[NP:]
    i = pl.program_id(0)

    @pl.when(i == 0)
    def _gather_and_hidden():
        for j in range(CTX):
            off = pl.multiple_of(idx_ref[j] // 128 * 128, 128)
            pltpu.make_async_copy(embt_hbm.at[:, pl.ds(off, 128)],
                                  embt_s.at[j], sem).start()
        for j in range(CTX):
            off = pl.multiple_of(idx_ref[j] // 128 * 128, 128)
            pltpu.make_async_copy(embt_hbm.at[:, pl.ds(off, 128)],
                                  embt_s.at[j], sem).wait()
        lanes = jax.lax.broadcasted_iota(jnp.int32, (1, 128), 1)
        acc = jnp.zeros((1, HID), jnp.float32)
        for j in range(CTX):
            onehot = (lanes == idx_ref[j] % 128).astype(jnp.float32)
            emb_j = jax.lax.dot_general(
                onehot, embt_s[j], _NT,
                preferred_element_type=jnp.float32)
            acc = acc + jnp.dot(emb_j, w1_ref[j],
                                preferred_element_type=jnp.float32)
        hid_s[...] = jnp.maximum(acc + b1_ref[...], 0.0)

    hid = hid_s[...]
    for p, w2_ref in enumerate(w2_refs):
        b = i * NP + p
        logits = jax.lax.dot_general(
            hid, w2_ref[...], _NT,
            preferred_element_type=jnp.float32) + b2_ref[b]
        out_ref[b, :] = logits[0]

    @pl.when(i == NSTEP - 1)
    def _log_softmax():
        out_ref[NBLK - 1, VOCAB - (NBLK - 1) * BN:] = jnp.full(
            (VPAD - VOCAB,), NEG, jnp.float32)
        x = out_ref[...]
        m = jnp.max(x)
        lse = m + jnp.log(jnp.sum(jnp.exp(x - m)))
        out_ref[...] = x - lse


@jax.jit
def _fwd(idx, embt, w1, b1, w2t, b2pad):
    grid_spec = pltpu.PrefetchScalarGridSpec(
        num_scalar_prefetch=1,
        grid=(NSTEP,),
        in_specs=[
            pl.BlockSpec(memory_space=pl.ANY),
            pl.BlockSpec((CTX, EMB, HID), lambda i, ref: (0, 0, 0)),
            pl.BlockSpec((1, HID), lambda i, ref: (0, 0)),
            *[pl.BlockSpec(
                  (BN, HID),
                  functools.partial(
                      lambda p, i, ref: (jnp.minimum(NP * i + p, LASTB), 0),
                      p))
              for p in range(NP)],
            pl.BlockSpec((NBLK, BN), lambda i, ref: (0, 0)),
        ],
        out_specs=pl.BlockSpec((NBLK, BN), lambda i, ref: (0, 0)),
        scratch_shapes=[
            pltpu.VMEM((CTX, EMB, 128), jnp.float32),
            pltpu.VMEM((1, HID), jnp.float32),
            pltpu.SemaphoreType.DMA,
        ],
    )
    return pl.pallas_call(
        _fwd_body,
        grid_spec=grid_spec,
        out_shape=jax.ShapeDtypeStruct((NBLK, BN), jnp.float32),
        compiler_params=pltpu.CompilerParams(
            dimension_semantics=("arbitrary",),
        ),
    )(idx, embt, w1, b1, *([w2t] * NP), b2pad)


def kernel(context_words, emb_table, W1, b1, W2, b2):
    idx = context_words.astype(jnp.int32)
    embt = emb_table.T
    w2t = W2.T
    w1 = W1.reshape(CTX, EMB, HID)
    b1r = b1.reshape(1, HID)
    b2pad = jnp.pad(b2, (0, VPAD - VOCAB)).reshape(NBLK, BN)
    out = _fwd(idx, embt, w1, b1r, w2t, b2pad)
    return out.reshape(1, VPAD)[:, :VOCAB]
